# baseline (speedup 1.0000x reference)
import functools

import jax
import jax.numpy as jnp
from jax import lax
from jax.experimental import pallas as pl
from jax.experimental.pallas import tpu as pltpu
from jax.experimental.pallas import tpu_sc as plsc

NC = 2
NS = 16
NW = NC * NS
CH = 128

_mesh = lambda: plsc.VectorSubcoreMesh(core_axis_name="c", subcore_axis_name="s")


def _deg_body(npad, t0, t1, dst_hbm, zeros_hbm, ones_hbm, out_hbm,
              dstv, zbuf, onesv, acc):
  c = lax.axis_index("c")
  s = lax.axis_index("s")
  wid = c * NS + s
  t_lim = lax.select(c == 0, t0, t1)
  rows = npad // NS
  pltpu.sync_copy(dst_hbm.at[wid], dstv)
  pltpu.sync_copy(zeros_hbm, zbuf)
  pltpu.sync_copy(zbuf, acc.at[pl.ds(s * rows, rows)])
  pltpu.sync_copy(ones_hbm, onesv)
  plsc.subcore_barrier()

  def body(t, carry):
    pltpu.sync_copy(onesv, acc.at[dstv.at[t]], add=True)
    return carry

  lax.fori_loop(0, t_lim, body, 0)
  plsc.subcore_barrier()
  pltpu.sync_copy(acc.at[pl.ds(s * rows, rows)],
                  out_hbm.at[c, pl.ds(s * rows, rows)])


def _msg_body(npad, t0, t1, h, src_hbm, dst_hbm, y_hbm, zeros_hbm, out_hbm,
              srcv, dstv, zbuf, rowsv, acc):
  c = lax.axis_index("c")
  s = lax.axis_index("s")
  wid = c * NS + s
  t_lim = lax.select(c == 0, t0, t1)
  rows = npad // NS
  pltpu.sync_copy(src_hbm.at[wid], srcv)
  pltpu.sync_copy(dst_hbm.at[wid], dstv)
  pltpu.sync_copy(zeros_hbm, zbuf)
  pltpu.sync_copy(zbuf, acc.at[pl.ds(s * rows, rows)])
  plsc.subcore_barrier()

  def body(t, carry):
    pltpu.sync_copy(y_hbm.at[srcv.at[t]], rowsv)
    pltpu.sync_copy(rowsv, acc.at[dstv.at[t]], add=True)
    return carry

  lax.fori_loop(0, t_lim, body, 0)
  plsc.subcore_barrier()
  pltpu.sync_copy(acc.at[pl.ds(s * rows, rows)],
                  out_hbm.at[c, pl.ds(s * rows, rows)])


def _prep_body(xp_ref, we_ref, degp_ref, y_ref, dinv_ref):
  deg = degp_ref[0] + degp_ref[1] + 1.0
  dinv = lax.rsqrt(jnp.maximum(deg[:, 0:1], 1e-12))
  xw = jnp.dot(xp_ref[...], we_ref[...], preferred_element_type=jnp.float32)
  y_ref[...] = xw * dinv
  dinv_ref[...] = dinv


def _dec_body(p_ref, y_ref, dinv_ref, benc_ref, w_ref, bdec_ref, o_ref,
              enc_ref):
  j = pl.program_id(1)

  @pl.when(j == 0)
  def _():
    enc_ref[...] = ((p_ref[0] + p_ref[1] + y_ref[...]) * dinv_ref[...]
                    + benc_ref[...]) * 0.5

  z = jnp.dot(enc_ref[...], w_ref[...], preferred_element_type=jnp.float32)
  o_ref[...] = 0.5 + 0.5 * jnp.tanh(z + bdec_ref[...])


def kernel(x, edge_index, W_enc, b_enc, W_dec, b_dec):
  n, d = x.shape
  h = W_enc.shape[1]
  out_dim = W_dec.shape[1]
  e = edge_index.shape[1]

  npad = ((n + 1 + 511) // 512) * 512
  rows = npad // NS

  n_chunks = (e + CH - 1) // CH
  tot = ((n_chunks + NS - 1) // NS + 1)
  t0 = int(round(tot * 0.586))
  t1 = tot - t0
  e0 = NS * t0 * CH
  pad_total = NS * (t0 + t1) * CH - e

  def part(row):
    flat = jnp.concatenate([row, jnp.full((pad_total,), n, jnp.int32)])
    a = flat[:e0].reshape(NS, t0, CH)
    b = flat[e0:].reshape(NS, t1, CH)
    b = jnp.pad(b, ((0, 0), (0, t0 - t1), (0, 0)), constant_values=n)
    return jnp.concatenate([a, b], axis=0)

  src3 = part(edge_index[0])
  dst3 = part(edge_index[1])
  t_chunks = t0

  zeros16 = jnp.zeros((rows, 16), jnp.float32)
  ones16 = jnp.ones((CH, 16), jnp.float32)
  zerosh = jnp.zeros((rows, h), jnp.float32)

  deg_call = pl.kernel(
      functools.partial(_deg_body, npad, t0, t1),
      out_type=jax.ShapeDtypeStruct((NC, npad, 16), jnp.float32),
      mesh=_mesh(),
      compiler_params=pltpu.CompilerParams(use_tc_tiling_on_sc=False),
      scratch_types=[
          pltpu.VMEM((t_chunks, CH), jnp.int32),
          pltpu.VMEM((rows, 16), jnp.float32),
          pltpu.VMEM((CH, 16), jnp.float32),
          pltpu.VMEM_SHARED((npad, 16), jnp.float32),
      ],
  )
  degp = deg_call(dst3, zeros16, ones16)

  xp = jnp.pad(x, ((0, npad - n), (0, 0)))
  bm_p = 1024
  prep_call = pl.pallas_call(
      _prep_body,
      grid=(npad // bm_p,),
      in_specs=[
          pl.BlockSpec((bm_p, d), lambda i: (i, 0)),
          pl.BlockSpec((d, h), lambda i: (0, 0)),
          pl.BlockSpec((NC, bm_p, 16), lambda i: (0, i, 0)),
      ],
      out_specs=[
          pl.BlockSpec((bm_p, h), lambda i: (i, 0)),
          pl.BlockSpec((bm_p, 1), lambda i: (i, 0)),
      ],
      out_shape=[
          jax.ShapeDtypeStruct((npad, h), jnp.float32),
          jax.ShapeDtypeStruct((npad, 1), jnp.float32),
      ],
  )
  y, dinv = prep_call(xp, W_enc, degp)

  msg_call = pl.kernel(
      functools.partial(_msg_body, npad, t0, t1, h),
      out_type=jax.ShapeDtypeStruct((NC, npad, h), jnp.float32),
      mesh=_mesh(),
      compiler_params=pltpu.CompilerParams(use_tc_tiling_on_sc=False),
      scratch_types=[
          pltpu.VMEM((t_chunks, CH), jnp.int32),
          pltpu.VMEM((t_chunks, CH), jnp.int32),
          pltpu.VMEM((rows, h), jnp.float32),
          pltpu.VMEM((CH, h), jnp.float32),
          pltpu.VMEM_SHARED((npad, h), jnp.float32),
      ],
  )
  p = msg_call(src3, dst3, y, zerosh)

  bm, bn = 1000, 1024
  dec_call = pl.pallas_call(
      _dec_body,
      grid=(n // bm, pl.cdiv(out_dim, bn)),
      in_specs=[
          pl.BlockSpec((NC, bm, h), lambda i, j: (0, i, 0)),
          pl.BlockSpec((bm, h), lambda i, j: (i, 0)),
          pl.BlockSpec((bm, 1), lambda i, j: (i, 0)),
          pl.BlockSpec((1, h), lambda i, j: (0, 0)),
          pl.BlockSpec((h, bn), lambda i, j: (0, j)),
          pl.BlockSpec((1, bn), lambda i, j: (0, j)),
      ],
      out_specs=pl.BlockSpec((bm, bn), lambda i, j: (i, j)),
      out_shape=jax.ShapeDtypeStruct((n, out_dim), jnp.float32),
      scratch_shapes=[pltpu.VMEM((bm, h), jnp.float32)],
  )
  return dec_call(p, y, dinv, b_enc.reshape(1, h), W_dec,
                  (0.5 * b_dec).reshape(1, out_dim))


# speedup vs baseline: 1.1477x; 1.1477x over previous
import functools

import jax
import jax.numpy as jnp
from jax import lax
from jax.experimental import pallas as pl
from jax.experimental.pallas import tpu as pltpu
from jax.experimental.pallas import tpu_sc as plsc

NC = 2
NS = 16
NW = NC * NS
CH = 128

_mesh = lambda: plsc.VectorSubcoreMesh(core_axis_name="c", subcore_axis_name="s")


def _deg_body(npad, t_chunks, dst_hbm, zeros_hbm, ones_hbm, out_hbm,
              dstv, zbuf, onesv, acc):
  c = lax.axis_index("c")
  s = lax.axis_index("s")
  wid = c * NS + s
  rows = npad // NS
  pltpu.sync_copy(dst_hbm.at[wid], dstv)
  pltpu.sync_copy(zeros_hbm, zbuf)
  pltpu.sync_copy(zbuf, acc.at[pl.ds(s * rows, rows)])
  pltpu.sync_copy(ones_hbm, onesv)
  plsc.subcore_barrier()

  def body(t, carry):
    pltpu.sync_copy(onesv, acc.at[dstv.at[t]], add=True)
    return carry

  lax.fori_loop(0, t_chunks, body, 0)
  plsc.subcore_barrier()
  pltpu.sync_copy(acc.at[pl.ds(s * rows, rows)],
                  out_hbm.at[c, pl.ds(s * rows, rows)])


def _msg_body(npad, t_chunks, h, src_hbm, dst_hbm, y_hbm, zeros_hbm, out_hbm,
              srcv, dstv, zbuf, rowsv, acc):
  c = lax.axis_index("c")
  s = lax.axis_index("s")
  wid = c * NS + s
  rows = npad // NS
  pltpu.sync_copy(src_hbm.at[wid], srcv)
  pltpu.sync_copy(dst_hbm.at[wid], dstv)
  pltpu.sync_copy(zeros_hbm, zbuf)
  pltpu.sync_copy(zbuf, acc.at[pl.ds(s * rows, rows)])
  plsc.subcore_barrier()

  def body(t, carry):
    pltpu.sync_copy(y_hbm.at[srcv.at[t]], rowsv)
    pltpu.sync_copy(rowsv, acc.at[dstv.at[t]], add=True)
    return carry

  lax.fori_loop(0, t_chunks, body, 0)
  plsc.subcore_barrier()
  pltpu.sync_copy(acc.at[pl.ds(s * rows, rows)],
                  out_hbm.at[c, pl.ds(s * rows, rows)])


def _prep_body(xp_ref, we_ref, degp_ref, y_ref, dinv_ref):
  deg = degp_ref[0] + degp_ref[1] + 1.0
  dinv = lax.rsqrt(jnp.maximum(deg[:, 0:1], 1e-12))
  xw = jnp.dot(xp_ref[...], we_ref[...], preferred_element_type=jnp.float32)
  y_ref[...] = xw * dinv
  dinv_ref[...] = dinv


def _dec_body(p_ref, y_ref, dinv_ref, benc_ref, w_ref, bdec_ref, o_ref,
              enc_ref):
  j = pl.program_id(1)

  @pl.when(j == 0)
  def _():
    enc_ref[...] = ((p_ref[0] + p_ref[1] + y_ref[...]) * dinv_ref[...]
                    + benc_ref[...]) * 0.5

  z = jnp.dot(enc_ref[...], w_ref[...], preferred_element_type=jnp.float32)
  o_ref[...] = 0.5 + 0.5 * jnp.tanh(z + bdec_ref[...])


def kernel(x, edge_index, W_enc, b_enc, W_dec, b_dec):
  n, d = x.shape
  h = W_enc.shape[1]
  out_dim = W_dec.shape[1]
  e = edge_index.shape[1]

  npad = ((n + 1 + 511) // 512) * 512
  rows = npad // NS

  ch = 0
  for cand in range(CH, 99, -1):
    if e % (NW * cand) == 0:
      ch = cand
      break
  if ch:
    t_chunks = e // (NW * ch)
    src3 = edge_index[0].reshape(NW, t_chunks, ch)
    dst3 = edge_index[1].reshape(NW, t_chunks, ch)
  else:
    ch = CH
    ep = -(-e // (NW * ch)) * (NW * ch)
    t_chunks = ep // (NW * ch)
    padv = jnp.full((ep - e,), n, jnp.int32)
    src3 = jnp.concatenate([edge_index[0], padv]).reshape(NW, t_chunks, ch)
    dst3 = jnp.concatenate([edge_index[1], padv]).reshape(NW, t_chunks, ch)

  zeros16 = jnp.zeros((rows, 16), jnp.float32)
  ones16 = jnp.ones((ch, 16), jnp.float32)
  zerosh = jnp.zeros((rows, h), jnp.float32)

  deg_call = pl.kernel(
      functools.partial(_deg_body, npad, t_chunks),
      out_type=jax.ShapeDtypeStruct((NC, npad, 16), jnp.float32),
      mesh=_mesh(),
      compiler_params=pltpu.CompilerParams(use_tc_tiling_on_sc=False),
      scratch_types=[
          pltpu.VMEM((t_chunks, ch), jnp.int32),
          pltpu.VMEM((rows, 16), jnp.float32),
          pltpu.VMEM((ch, 16), jnp.float32),
          pltpu.VMEM_SHARED((npad, 16), jnp.float32),
      ],
  )
  degp = deg_call(dst3, zeros16, ones16)

  xp = jnp.pad(x, ((0, npad - n), (0, 0)))
  bm_p = 1024
  prep_call = pl.pallas_call(
      _prep_body,
      grid=(npad // bm_p,),
      in_specs=[
          pl.BlockSpec((bm_p, d), lambda i: (i, 0)),
          pl.BlockSpec((d, h), lambda i: (0, 0)),
          pl.BlockSpec((NC, bm_p, 16), lambda i: (0, i, 0)),
      ],
      out_specs=[
          pl.BlockSpec((bm_p, h), lambda i: (i, 0)),
          pl.BlockSpec((bm_p, 1), lambda i: (i, 0)),
      ],
      out_shape=[
          jax.ShapeDtypeStruct((npad, h), jnp.float32),
          jax.ShapeDtypeStruct((npad, 1), jnp.float32),
      ],
  )
  y, dinv = prep_call(xp, W_enc, degp)

  msg_call = pl.kernel(
      functools.partial(_msg_body, npad, t_chunks, h),
      out_type=jax.ShapeDtypeStruct((NC, npad, h), jnp.float32),
      mesh=_mesh(),
      compiler_params=pltpu.CompilerParams(use_tc_tiling_on_sc=False),
      scratch_types=[
          pltpu.VMEM((t_chunks, ch), jnp.int32),
          pltpu.VMEM((t_chunks, ch), jnp.int32),
          pltpu.VMEM((rows, h), jnp.float32),
          pltpu.VMEM((ch, h), jnp.float32),
          pltpu.VMEM_SHARED((npad, h), jnp.float32),
      ],
  )
  p = msg_call(src3, dst3, y, zerosh)

  bm, bn = 1000, 1024
  dec_call = pl.pallas_call(
      _dec_body,
      grid=(n // bm, pl.cdiv(out_dim, bn)),
      in_specs=[
          pl.BlockSpec((NC, bm, h), lambda i, j: (0, i, 0)),
          pl.BlockSpec((bm, h), lambda i, j: (i, 0)),
          pl.BlockSpec((bm, 1), lambda i, j: (i, 0)),
          pl.BlockSpec((1, h), lambda i, j: (0, 0)),
          pl.BlockSpec((h, bn), lambda i, j: (0, j)),
          pl.BlockSpec((1, bn), lambda i, j: (0, j)),
      ],
      out_specs=pl.BlockSpec((bm, bn), lambda i, j: (i, j)),
      out_shape=jax.ShapeDtypeStruct((n, out_dim), jnp.float32),
      scratch_shapes=[pltpu.VMEM((bm, h), jnp.float32)],
  )
  return dec_call(p, y, dinv, b_enc.reshape(1, h), W_dec,
                  (0.5 * b_dec).reshape(1, out_dim))
